# Initial kernel scaffold; baseline (speedup 1.0000x reference)
import jax
import jax.numpy as jnp
from jax.experimental import pallas as pl


def kernel(neigh_rel, neigh_ent, e1_degrees, entity_embds, relation_embds, W_agg, b_agg, W_self, b_self):
    raise NotImplementedError("write your pallas kernel here")


# speedup vs baseline: 3.0377x; 3.0377x over previous
import functools

import jax
import jax.numpy as jnp
from jax import lax
from jax.experimental import pallas as pl
from jax.experimental.pallas import tpu as pltpu
from jax.experimental.pallas import tpu_sc as plsc

N = 50000
R = 474
D = 128
MAXN = 10

NC = 2
NS = 16
NW = NC * NS

CH = 8
PER_W = 1568
NK = PER_W // CH
NPAD = PER_W * NW


def _sc_segment_sums(idx_rel_flat, idx_ent_flat, rel_tbl, ent_tbl):
    mesh = plsc.VectorSubcoreMesh(core_axis_name="c", subcore_axis_name="s")

    @functools.partial(
        pl.kernel,
        out_type=jax.ShapeDtypeStruct((NPAD, 2 * D), jnp.float32),
        mesh=mesh,
        scratch_types=[
            pltpu.VMEM((CH * MAXN,), jnp.int32),
            pltpu.VMEM((CH * MAXN,), jnp.int32),
            pltpu.VMEM((CH * MAXN, D), jnp.float32),
            pltpu.VMEM((CH * MAXN, D), jnp.float32),
            pltpu.VMEM((CH, 2 * D), jnp.float32),
            pltpu.SemaphoreType.DMA,
            pltpu.SemaphoreType.DMA,
        ],
    )
    def k(idx_rel_hbm, idx_ent_hbm, rel_hbm, ent_hbm, out_hbm,
          idx_r, idx_e, rows_r, rows_e, out_v, sem_r, sem_e):
        wid = lax.axis_index("s") * NC + lax.axis_index("c")

        def chunk_body(ki, carry):
            b = wid * PER_W + ki * CH
            pltpu.sync_copy(idx_rel_hbm.at[pl.ds(b * MAXN, CH * MAXN)], idx_r)
            pltpu.sync_copy(idx_ent_hbm.at[pl.ds(b * MAXN, CH * MAXN)], idx_e)
            cp_r = pltpu.async_copy(rel_hbm.at[idx_r], rows_r, sem_r)
            cp_e = pltpu.async_copy(ent_hbm.at[idx_e], rows_e, sem_e)
            cp_r.wait()
            cp_e.wait()

            def ent_body(e, carry2):
                base = e * MAXN
                for c in range(D // 16):
                    col = c * 16
                    sr = rows_r[base, pl.ds(col, 16)]
                    se = rows_e[base, pl.ds(col, 16)]
                    for r in range(1, MAXN):
                        sr = sr + rows_r[base + r, pl.ds(col, 16)]
                        se = se + rows_e[base + r, pl.ds(col, 16)]
                    out_v[e, pl.ds(col, 16)] = sr
                    out_v[e, pl.ds(D + col, 16)] = se
                return carry2

            lax.fori_loop(0, CH, ent_body, 0)
            pltpu.sync_copy(out_v, out_hbm.at[pl.ds(b, CH)])
            return carry

        lax.fori_loop(0, NK, chunk_body, 0)

    return k(idx_rel_flat, idx_ent_flat, rel_tbl, ent_tbl)


def _tc_head(sums_pad, entity_embds, deg_col, W_agg, W_self, b_agg, b_self):
    RB = 1000
    grid = (N // RB,)

    def body(sums_ref, ent_ref, deg_ref, wa_ref, ws_ref, ba_ref, bs_ref, out_ref):
        agg = jnp.dot(sums_ref[...], wa_ref[...],
                      preferred_element_type=jnp.float32)
        agg = (agg + MAXN * ba_ref[...]) / deg_ref[...]
        ws = ws_ref[...]
        x = jnp.dot(ent_ref[...], ws[:D], preferred_element_type=jnp.float32)
        x = x + jnp.dot(agg, ws[D:], preferred_element_type=jnp.float32)
        out_ref[...] = jnp.tanh(x + bs_ref[...])

    return pl.pallas_call(
        body,
        grid=grid,
        in_specs=[
            pl.BlockSpec((RB, 2 * D), lambda i: (i, 0)),
            pl.BlockSpec((RB, D), lambda i: (i, 0)),
            pl.BlockSpec((RB, 1), lambda i: (i, 0)),
            pl.BlockSpec((2 * D, D), lambda i: (0, 0)),
            pl.BlockSpec((2 * D, D), lambda i: (0, 0)),
            pl.BlockSpec((1, D), lambda i: (0, 0)),
            pl.BlockSpec((1, D), lambda i: (0, 0)),
        ],
        out_specs=pl.BlockSpec((RB, D), lambda i: (i, 0)),
        out_shape=jax.ShapeDtypeStruct((N, D), jnp.float32),
    )(sums_pad, entity_embds, deg_col, W_agg, W_self, b_agg, b_self)


def kernel(neigh_rel, neigh_ent, e1_degrees, entity_embds, relation_embds,
           W_agg, b_agg, W_self, b_self):
    pad = NPAD - N
    idx_rel = jnp.pad(neigh_rel.astype(jnp.int32), ((0, pad), (0, 0))).reshape(-1)
    idx_ent = jnp.pad(neigh_ent.astype(jnp.int32), ((0, pad), (0, 0))).reshape(-1)

    sums = _sc_segment_sums(idx_rel, idx_ent, relation_embds, entity_embds)

    return _tc_head(sums, entity_embds, e1_degrees.reshape(-1, 1),
                    W_agg, W_self, b_agg.reshape(1, D), b_self.reshape(1, D))
